# baseline (speedup 1.0000x reference)
import functools

import jax
import jax.numpy as jnp
from jax import lax
from jax.experimental import pallas as pl
from jax.experimental.pallas import tpu as pltpu
from jax.experimental.pallas import tpu_sc as plsc

_NW = 32
_CH = 128


def _sc_edge_phase(node_states, a_in, src2, dst2, n_nodes):
    nchunk, ch = src2.shape
    d = node_states.shape[1]

    base_c = nchunk // _NW
    rem = nchunk % _NW
    maxc = base_c + (1 if rem else 0)
    zb = 80
    nzc = n_nodes // zb
    mesh = plsc.VectorSubcoreMesh(core_axis_name="c", subcore_axis_name="s")

    @functools.partial(
        pl.kernel,
        mesh=mesh,
        out_type=[
            jax.ShapeDtypeStruct((2 * n_nodes, d), jnp.float32),
            jax.ShapeDtypeStruct((2 * n_nodes, d), jnp.float32),
        ],
        compiler_params=pltpu.CompilerParams(use_tc_tiling_on_sc=False),
        scratch_types=[
            pltpu.VMEM((maxc, ch), jnp.int32),
            pltpu.VMEM((maxc, ch), jnp.int32),
            pltpu.VMEM((ch, d), jnp.float32),
            pltpu.VMEM((ch, d), jnp.float32),
            pltpu.VMEM((ch, d, d), jnp.float32),
            pltpu.VMEM((ch, d, d), jnp.float32),
            pltpu.VMEM((ch, d), jnp.float32),
            pltpu.VMEM((ch, d), jnp.float32),
            pltpu.VMEM((ch, d), jnp.float32),
            pltpu.VMEM((zb, d), jnp.float32),
            pltpu.VMEM_SHARED((n_nodes, d), jnp.float32),
            pltpu.VMEM_SHARED((n_nodes, d), jnp.float32),
            pltpu.SemaphoreType.DMA,
            pltpu.SemaphoreType.DMA,
            pltpu.SemaphoreType.DMA,
            pltpu.SemaphoreType.DMA,
            pltpu.SemaphoreType.DMA,
        ],
    )
    def k(ns_hbm, a_hbm, src_hbm, dst_hbm, sums_hbm, cnts_hbm,
          sidx, didx, xb0, xb1, ab0, ab1, mb0, mb1, ones_v, zero_v,
          acc_sh, cnt_sh, sl0, sl1, sm0, sm1, sem_c):
        core = lax.axis_index("c")
        sid = lax.axis_index("s")
        wid = sid * 2 + core
        startc = base_c * wid + jnp.minimum(wid, rem)
        x_buf, a_buf, m_buf = (xb0, xb1), (ab0, ab1), (mb0, mb1)
        sem_ld, sem_m = (sl0, sl1), (sm0, sm1)

        @pl.loop(0, ch)
        def _(i):
            ones_v[i] = jnp.ones((d,), jnp.float32)

        @pl.loop(0, zb)
        def _(i):
            zero_v[i] = jnp.zeros((d,), jnp.float32)

        @pl.loop(sid, nzc, step=16)
        def _(c):
            pltpu.sync_copy(zero_v, acc_sh.at[pl.ds(c * zb, zb)])
            pltpu.sync_copy(zero_v, cnt_sh.at[pl.ds(c * zb, zb)])

        def load_idx(cnt):
            pltpu.sync_copy(src_hbm.at[pl.ds(startc, cnt)],
                            sidx.at[pl.ds(0, cnt)])
            pltpu.sync_copy(dst_hbm.at[pl.ds(startc, cnt)],
                            didx.at[pl.ds(0, cnt)])

        if rem:
            @pl.when(wid < rem)
            def _():
                load_idx(base_c + 1)

            @pl.when(wid >= rem)
            def _():
                load_idx(base_c)
        else:
            load_idx(base_c)

        plsc.subcore_barrier()

        def fire_loads(j, b):
            pltpu.async_copy(ns_hbm.at[sidx.at[j]],
                             x_buf[b], sem_ld[b])
            pltpu.async_copy(a_hbm.at[pl.ds((startc + j) * ch, ch)],
                             a_buf[b], sem_ld[b])

        def wait_loads(b):
            pltpu.make_async_copy(ns_hbm.at[pl.ds(0, ch)],
                                  x_buf[b], sem_ld[b]).wait()
            pltpu.make_async_copy(a_hbm.at[pl.ds(0, ch)],
                                  a_buf[b], sem_ld[b]).wait()

        def drain_add(b):
            pltpu.make_async_copy(ns_hbm.at[pl.ds(0, ch)],
                                  m_buf[b], sem_m[b]).wait()

        def item(j, b, fire_next, has_prev):
            if fire_next is not None:
                @pl.when(fire_next)
                def _():
                    fire_loads(j + 1, 1 - b)
            wait_loads(b)
            if has_prev is True:
                drain_add(b)
            elif has_prev is not None:
                @pl.when(has_prev)
                def _():
                    drain_add(b)
            xb, ab, mb = x_buf[b], a_buf[b], m_buf[b]

            @pl.loop(0, ch)
            def _(e):
                xv = xb[e]
                m = ab[e, 0] * xv[0]
                for dd in range(1, d):
                    m = m + ab[e, dd] * xv[dd]
                mb[e] = m

            pltpu.async_copy(mb, acc_sh.at[didx.at[j]],
                             sem_m[b], add=True)
            pltpu.async_copy(ones_v, cnt_sh.at[didx.at[j]],
                             sem_c, add=True)

        def flow(cnt):
            fire_loads(0, 0)
            pairs = cnt // 2

            @pl.loop(0, pairs)
            def _(jj):
                j = 2 * jj
                item(j, 0, j + 1 < cnt, j >= 2)
                item(j + 1, 1, j + 2 < cnt, j + 1 >= 3)

            if cnt % 2:
                item(cnt - 1, 0, None, True if cnt - 1 >= 2 else None)
            if cnt >= 2:
                drain_add(0)
                drain_add(1)
            elif cnt == 1:
                drain_add(0)

            @pl.loop(0, cnt)
            def _(j):
                pltpu.make_async_copy(ns_hbm.at[pl.ds(0, ch)],
                                      ones_v, sem_c).wait()

        if rem:
            @pl.when(wid < rem)
            def _():
                flow(base_c + 1)

            @pl.when(wid >= rem)
            def _():
                flow(base_c)
        else:
            flow(base_c)

        plsc.subcore_barrier()

        @pl.loop(sid, nzc, step=16)
        def _(c):
            pltpu.sync_copy(acc_sh.at[pl.ds(c * zb, zb)],
                            sums_hbm.at[pl.ds(core * n_nodes + c * zb, zb)])
            pltpu.sync_copy(cnt_sh.at[pl.ds(c * zb, zb)],
                            cnts_hbm.at[pl.ds(core * n_nodes + c * zb, zb)])

    return k(node_states, a_in, src2, dst2)


def _tc_gru(node_states, sums, cnts, w_ih, w_hh, b_ih, b_hh):
    n, d = node_states.shape
    blk = 2000
    grid = n // blk
    nb = n // blk

    def body(h_ref, s0_ref, s1_ref, c0_ref, c1_ref,
             wih_ref, whh_ref, bih_ref, bhh_ref, o_ref):
        s = s0_ref[...] + s1_ref[...]
        c = c0_ref[...] + c1_ref[...]
        mean = s / jnp.maximum(c, 1.0)
        h = h_ref[...]
        dims = (((1,), (1,)), ((), ()))
        gx = lax.dot_general(mean, wih_ref[...], dims,
                             precision=lax.Precision.HIGHEST) + bih_ref[0]
        gh = lax.dot_general(h, whh_ref[...], dims,
                             precision=lax.Precision.HIGHEST) + bhh_ref[0]
        r = jax.nn.sigmoid(gx[:, :d] + gh[:, :d])
        z = jax.nn.sigmoid(gx[:, d:2 * d] + gh[:, d:2 * d])
        nn = jnp.tanh(gx[:, 2 * d:] + r * gh[:, 2 * d:])
        o_ref[...] = (1.0 - z) * nn + z * h

    return pl.pallas_call(
        body,
        grid=(grid,),
        in_specs=[
            pl.BlockSpec((blk, d), lambda i: (i, 0)),
            pl.BlockSpec((blk, d), lambda i: (i, 0)),
            pl.BlockSpec((blk, d), lambda i, _nb=nb: (i + _nb, 0)),
            pl.BlockSpec((blk, d), lambda i: (i, 0)),
            pl.BlockSpec((blk, d), lambda i, _nb=nb: (i + _nb, 0)),
            pl.BlockSpec((3 * d, d), lambda i: (0, 0)),
            pl.BlockSpec((3 * d, d), lambda i: (0, 0)),
            pl.BlockSpec((1, 3 * d), lambda i: (0, 0)),
            pl.BlockSpec((1, 3 * d), lambda i: (0, 0)),
        ],
        out_specs=pl.BlockSpec((blk, d), lambda i: (i, 0)),
        out_shape=jax.ShapeDtypeStruct((n, d), jnp.float32),
    )(node_states, sums, sums, cnts, cnts,
      w_ih, w_hh, b_ih.reshape(1, 3 * d), b_hh.reshape(1, 3 * d))


def kernel(node_states, edge_index, a_in, w_ih, w_hh, b_ih, b_hh):
    e_total = edge_index.shape[0]
    n, d = node_states.shape
    src2 = edge_index[:, 0].reshape(e_total // _CH, _CH)
    dst2 = edge_index[:, 1].reshape(e_total // _CH, _CH)
    sums, cnts = _sc_edge_phase(node_states, a_in, src2, dst2, n)
    return _tc_gru(node_states, sums, cnts, w_ih, w_hh, b_ih, b_hh)


# speedup vs baseline: 5.0073x; 5.0073x over previous
import functools

import jax
import jax.numpy as jnp
from jax import lax
from jax.experimental import pallas as pl
from jax.experimental.pallas import tpu as pltpu
from jax.experimental.pallas import tpu_sc as plsc

_NW = 32
_CH = 128


def _transpose_chunk(src_v, row0, dst_v, d, iotas):
    for dd in range(d):
        col = jnp.full((16,), dd, jnp.int32)
        for s in range(_CH // 16):
            v = plsc.load_gather(src_v, [row0 + iotas[s], col])
            dst_v[dd, pl.ds(s * 16, 16)] = v


def _sc_gather_t(node_states, src2, dst2, n_nodes):
    nchunk, ch = src2.shape
    e_total = nchunk * ch
    d = node_states.shape[1]
    base_c = nchunk // _NW
    rem = nchunk % _NW
    maxc = base_c + (1 if rem else 0)
    zb = 80
    nzc = n_nodes // zb
    mesh = plsc.VectorSubcoreMesh(core_axis_name="c", subcore_axis_name="s")

    @functools.partial(
        pl.kernel,
        mesh=mesh,
        out_type=[
            jax.ShapeDtypeStruct((d, e_total), jnp.float32),
            jax.ShapeDtypeStruct((2 * n_nodes, d), jnp.float32),
        ],
        compiler_params=pltpu.CompilerParams(use_tc_tiling_on_sc=False,
                                             needs_layout_passes=False),
        scratch_types=[
            pltpu.VMEM((maxc, ch), jnp.int32),
            pltpu.VMEM((maxc, ch), jnp.int32),
            pltpu.VMEM((maxc * ch, d), jnp.float32),
            pltpu.VMEM((d, ch), jnp.float32),
            pltpu.VMEM((d, ch), jnp.float32),
            pltpu.VMEM((ch, d), jnp.float32),
            pltpu.VMEM((zb, d), jnp.float32),
            pltpu.VMEM_SHARED((n_nodes, d), jnp.float32),
            pltpu.SemaphoreType.DMA,
            pltpu.SemaphoreType.DMA,
            pltpu.SemaphoreType.DMA,
            pltpu.SemaphoreType.DMA,
        ],
    )
    def k(ns_hbm, src_hbm, dst_hbm, xt_hbm, cnts_hbm,
          sidx, didx, rows_all, tb0, tb1, ones_v, zero_v, cnt_sh,
          sem_g, sw0, sw1, sem_c):
        core = lax.axis_index("c")
        sid = lax.axis_index("s")
        wid = sid * 2 + core
        startc = base_c * wid + jnp.minimum(wid, rem)
        t_buf = (tb0, tb1)
        sem_w = (sw0, sw1)
        iotas = [lax.iota(jnp.int32, 16) + (s * 16) for s in range(ch // 16)]

        @pl.loop(0, ch)
        def _(i):
            ones_v[i] = jnp.ones((d,), jnp.float32)

        @pl.loop(0, zb)
        def _(i):
            zero_v[i] = jnp.zeros((d,), jnp.float32)

        @pl.loop(sid, nzc, step=16)
        def _(c):
            pltpu.sync_copy(zero_v, cnt_sh.at[pl.ds(c * zb, zb)])

        def load_idx(cnt):
            pltpu.sync_copy(src_hbm.at[pl.ds(startc, cnt)],
                            sidx.at[pl.ds(0, cnt)])
            pltpu.sync_copy(dst_hbm.at[pl.ds(startc, cnt)],
                            didx.at[pl.ds(0, cnt)])

        if rem:
            @pl.when(wid < rem)
            def _():
                load_idx(base_c + 1)

            @pl.when(wid >= rem)
            def _():
                load_idx(base_c)
        else:
            load_idx(base_c)

        plsc.subcore_barrier()

        def flow(cnt):
            @pl.loop(0, cnt)
            def _(j):
                pltpu.async_copy(ns_hbm.at[sidx.at[j]],
                                 rows_all.at[pl.ds(j * ch, ch)], sem_g)

            @pl.loop(0, cnt)
            def _(j):
                pltpu.async_copy(ones_v, cnt_sh.at[didx.at[j]],
                                 sem_c, add=True)

            pltpu.make_async_copy(ns_hbm.at[pl.ds(0, cnt * ch)],
                                  rows_all.at[pl.ds(0, cnt * ch)],
                                  sem_g).wait()

            def titem(j, b, drain):
                if drain:
                    pltpu.make_async_copy(xt_hbm.at[:, pl.ds(0, ch)],
                                          t_buf[b], sem_w[b]).wait()
                _transpose_chunk(rows_all, j * ch, t_buf[b], d, iotas)
                pltpu.async_copy(
                    t_buf[b],
                    xt_hbm.at[:, pl.ds((startc + j) * ch, ch)], sem_w[b])

            titem(0, 0, False)
            if cnt >= 2:
                titem(1, 1, False)

                @pl.loop(2, cnt)
                def _(j):
                    jb = j % 2

                    @pl.when(jb == 0)
                    def _():
                        titem(j, 0, True)

                    @pl.when(jb == 1)
                    def _():
                        titem(j, 1, True)

            pltpu.make_async_copy(xt_hbm.at[:, pl.ds(0, ch)],
                                  t_buf[0], sem_w[0]).wait()
            if cnt >= 2:
                pltpu.make_async_copy(xt_hbm.at[:, pl.ds(0, ch)],
                                      t_buf[1], sem_w[1]).wait()

            @pl.loop(0, cnt)
            def _(j):
                pltpu.make_async_copy(ns_hbm.at[pl.ds(0, ch)],
                                      ones_v, sem_c).wait()

        if rem:
            @pl.when(wid < rem)
            def _():
                flow(base_c + 1)

            @pl.when(wid >= rem)
            def _():
                flow(base_c)
        else:
            flow(base_c)

        plsc.subcore_barrier()

        @pl.loop(sid, nzc, step=16)
        def _(c):
            pltpu.sync_copy(cnt_sh.at[pl.ds(c * zb, zb)],
                            cnts_hbm.at[pl.ds(core * n_nodes + c * zb, zb)])

    return k(node_states, src2, dst2)


def _tc_messages_t(x_t, a_t):
    d, _, e_total = a_t.shape
    be = 6400
    grid = e_total // be

    def body(x_ref, a_ref, o_ref):
        x = x_ref[...]
        acc = a_ref[0] * x[0]
        for dd in range(1, d):
            acc = acc + a_ref[dd] * x[dd]
        o_ref[...] = acc

    return pl.pallas_call(
        body,
        grid=(grid,),
        in_specs=[
            pl.BlockSpec((d, be), lambda i: (0, i)),
            pl.BlockSpec((d, d, be), lambda i: (0, 0, i)),
        ],
        out_specs=pl.BlockSpec((d, be), lambda i: (0, i)),
        out_shape=jax.ShapeDtypeStruct((d, e_total), jnp.float32),
    )(x_t, a_t)


def _sc_scatter_t(msg_t, dst2, n_nodes):
    nchunk, ch = dst2.shape
    d = msg_t.shape[0]
    base_c = nchunk // _NW
    rem = nchunk % _NW
    maxc = base_c + (1 if rem else 0)
    zb = 80
    nzc = n_nodes // zb
    mesh = plsc.VectorSubcoreMesh(core_axis_name="c", subcore_axis_name="s")

    @functools.partial(
        pl.kernel,
        mesh=mesh,
        out_type=jax.ShapeDtypeStruct((2 * n_nodes, d), jnp.float32),
        compiler_params=pltpu.CompilerParams(use_tc_tiling_on_sc=False,
                                             needs_layout_passes=False),
        scratch_types=[
            pltpu.VMEM((maxc, ch), jnp.int32),
            pltpu.VMEM((d, ch), jnp.float32),
            pltpu.VMEM((d, ch), jnp.float32),
            pltpu.VMEM((ch, d), jnp.float32),
            pltpu.VMEM((ch, d), jnp.float32),
            pltpu.VMEM((zb, d), jnp.float32),
            pltpu.VMEM_SHARED((n_nodes, d), jnp.float32),
            pltpu.SemaphoreType.DMA,
            pltpu.SemaphoreType.DMA,
            pltpu.SemaphoreType.DMA,
            pltpu.SemaphoreType.DMA,
        ],
    )
    def k(msg_hbm, dst_hbm, sums_hbm,
          didx, mt0, mt1, mc0, mc1, zero_v, acc_sh, sl0, sl1, sm0, sm1):
        core = lax.axis_index("c")
        sid = lax.axis_index("s")
        wid = sid * 2 + core
        startc = base_c * wid + jnp.minimum(wid, rem)
        mt_buf, mc_buf = (mt0, mt1), (mc0, mc1)
        sem_l, sem_m = (sl0, sl1), (sm0, sm1)
        iotas = [lax.iota(jnp.int32, 16) + (s * 16) for s in range(ch // 16)]

        @pl.loop(0, zb)
        def _(i):
            zero_v[i] = jnp.zeros((d,), jnp.float32)

        @pl.loop(sid, nzc, step=16)
        def _(c):
            pltpu.sync_copy(zero_v, acc_sh.at[pl.ds(c * zb, zb)])

        def load_idx(cnt):
            pltpu.sync_copy(dst_hbm.at[pl.ds(startc, cnt)],
                            didx.at[pl.ds(0, cnt)])

        if rem:
            @pl.when(wid < rem)
            def _():
                load_idx(base_c + 1)

            @pl.when(wid >= rem)
            def _():
                load_idx(base_c)
        else:
            load_idx(base_c)

        plsc.subcore_barrier()

        def fire_load(j, b):
            pltpu.async_copy(
                msg_hbm.at[:, pl.ds((startc + j) * ch, ch)],
                mt_buf[b], sem_l[b])

        def wait_load(b):
            pltpu.make_async_copy(msg_hbm.at[:, pl.ds(0, ch)],
                                  mt_buf[b], sem_l[b]).wait()

        def drain_add(b):
            pltpu.make_async_copy(sums_hbm.at[pl.ds(0, ch)],
                                  mc_buf[b], sem_m[b]).wait()

        iota_d = lax.iota(jnp.int32, 16)

        def mt_transpose(b):
            src, dst = mt_buf[b], mc_buf[b]

            @pl.loop(0, ch)
            def _(e):
                v = plsc.load_gather(src, [iota_d, jnp.full((16,), e,
                                                            jnp.int32)])
                dst[e] = v

        def item(j, b, fire_next, has_prev):
            if fire_next is not None:
                @pl.when(fire_next)
                def _():
                    fire_load(j + 1, 1 - b)
            wait_load(b)
            if has_prev is True:
                drain_add(b)
            elif has_prev is not None:
                @pl.when(has_prev)
                def _():
                    drain_add(b)
            mt_transpose(b)
            pltpu.async_copy(mc_buf[b], acc_sh.at[didx.at[j]],
                             sem_m[b], add=True)

        def flow(cnt):
            fire_load(0, 0)
            pairs = cnt // 2

            @pl.loop(0, pairs)
            def _(jj):
                j = 2 * jj
                item(j, 0, j + 1 < cnt, j >= 2)
                item(j + 1, 1, j + 2 < cnt, j + 1 >= 3)

            if cnt % 2:
                item(cnt - 1, 0, None, True if cnt - 1 >= 2 else None)
            if cnt >= 2:
                drain_add(0)
                drain_add(1)
            elif cnt == 1:
                drain_add(0)

        if rem:
            @pl.when(wid < rem)
            def _():
                flow(base_c + 1)

            @pl.when(wid >= rem)
            def _():
                flow(base_c)
        else:
            flow(base_c)

        plsc.subcore_barrier()

        @pl.loop(sid, nzc, step=16)
        def _(c):
            pltpu.sync_copy(acc_sh.at[pl.ds(c * zb, zb)],
                            sums_hbm.at[pl.ds(core * n_nodes + c * zb, zb)])

    return k(msg_t, dst2)


def _tc_gru(node_states, sums, cnts, w_ih, w_hh, b_ih, b_hh):
    n, d = node_states.shape
    blk = 2000
    grid = n // blk
    nb = n // blk

    def body(h_ref, s0_ref, s1_ref, c0_ref, c1_ref,
             wih_ref, whh_ref, bih_ref, bhh_ref, o_ref):
        s = s0_ref[...] + s1_ref[...]
        c = c0_ref[...] + c1_ref[...]
        mean = s / jnp.maximum(c, 1.0)
        h = h_ref[...]
        dims = (((1,), (1,)), ((), ()))
        gx = lax.dot_general(mean, wih_ref[...], dims,
                             precision=lax.Precision.HIGHEST) + bih_ref[0]
        gh = lax.dot_general(h, whh_ref[...], dims,
                             precision=lax.Precision.HIGHEST) + bhh_ref[0]
        r = jax.nn.sigmoid(gx[:, :d] + gh[:, :d])
        z = jax.nn.sigmoid(gx[:, d:2 * d] + gh[:, d:2 * d])
        nn = jnp.tanh(gx[:, 2 * d:] + r * gh[:, 2 * d:])
        o_ref[...] = (1.0 - z) * nn + z * h

    return pl.pallas_call(
        body,
        grid=(grid,),
        in_specs=[
            pl.BlockSpec((blk, d), lambda i: (i, 0)),
            pl.BlockSpec((blk, d), lambda i: (i, 0)),
            pl.BlockSpec((blk, d), lambda i, _nb=nb: (i + _nb, 0)),
            pl.BlockSpec((blk, d), lambda i: (i, 0)),
            pl.BlockSpec((blk, d), lambda i, _nb=nb: (i + _nb, 0)),
            pl.BlockSpec((3 * d, d), lambda i: (0, 0)),
            pl.BlockSpec((3 * d, d), lambda i: (0, 0)),
            pl.BlockSpec((1, 3 * d), lambda i: (0, 0)),
            pl.BlockSpec((1, 3 * d), lambda i: (0, 0)),
        ],
        out_specs=pl.BlockSpec((blk, d), lambda i: (i, 0)),
        out_shape=jax.ShapeDtypeStruct((n, d), jnp.float32),
    )(node_states, sums, sums, cnts, cnts,
      w_ih, w_hh, b_ih.reshape(1, 3 * d), b_hh.reshape(1, 3 * d))


def kernel(node_states, edge_index, a_in, w_ih, w_hh, b_ih, b_hh):
    e_total = edge_index.shape[0]
    n, d = node_states.shape
    src2 = edge_index[:, 0].reshape(e_total // _CH, _CH)
    dst2 = edge_index[:, 1].reshape(e_total // _CH, _CH)
    x_t, cnts = _sc_gather_t(node_states, src2, dst2, n)
    a_t = jnp.transpose(a_in, (1, 2, 0))
    msg_t = _tc_messages_t(x_t, a_t)
    sums = _sc_scatter_t(msg_t, dst2, n)
    return _tc_gru(node_states, sums, cnts, w_ih, w_hh, b_ih, b_hh)
